# baseline (speedup 1.0000x reference)
import functools

import jax
import jax.numpy as jnp
from jax import lax
from jax.experimental import pallas as pl
from jax.experimental.pallas import tpu as pltpu
from jax.experimental.pallas import tpu_sc as plsc

B = 64
S = 8
V = 100000
P = S + 1
NW = 32
BPW = B // NW
RPW = BPW * P
C = V // 2
U = 5
NSTEP = C // 16 // U
OUTW = 16
NEG_INF = float("-inf")
INT_MAX = 0x7FFFFFFF


def _xlane(x, idx):
    dn = lax.GatherDimensionNumbers(
        offset_dims=(), collapsed_slice_dims=(0,), start_index_map=(0,))
    return lax.gather(x, idx[:, None], dn, slice_sizes=(1,),
                      mode=lax.GatherScatterMode.PROMISE_IN_BOUNDS)


def _scan_chunk(buf, q, accs):
    @plsc.parallel_loop(0, NSTEP, step=1, unroll=8, carry=accs)
    def body(i, accs):
        gv = jnp.full((16,), q * NSTEP + i, dtype=jnp.int32)
        out = []
        for k in range(U):
            bv, bt = accs[2 * k], accs[2 * k + 1]
            v = buf[pl.ds((i * U + k) * 16, 16)]
            upd = v > bv
            bv = jnp.maximum(v, bv)
            bt = jnp.where(upd, gv, bt)
            out += [bv, bt]
        return tuple(out)

    return body


def _sc_kernel(logits_hbm, spec_hbm, out_hbm, buf0, buf1, spec_v, out_v,
               sem0, sem1):
    wid = lax.axis_index("s") * 2 + lax.axis_index("c")
    lanes = lax.iota(jnp.int32, 16)
    bufs = (buf0, buf1)
    sems = (sem0, sem1)

    def dma(j, q):
        base = (wid * RPW + j) * V + q * C
        return pltpu.make_async_copy(
            logits_hbm.at[pl.ds(base, C)], bufs[q], sems[q])

    pltpu.sync_copy(spec_hbm.at[pl.ds(wid * 16, 16)], spec_v.at[pl.ds(0, 16)])

    dma(0, 0).start()
    dma(0, 1).start()

    def row_body(j, ids):
        ids0, ids1 = ids
        accs = ()
        for _ in range(U):
            accs += (jnp.full((16,), NEG_INF, dtype=jnp.float32),
                     jnp.zeros((16,), dtype=jnp.int32))
        for q in range(2):
            dma(j, q).wait()
            accs = _scan_chunk(bufs[q], q, accs)

            @pl.when(j < RPW - 1)
            def _():
                dma(j + 1, q).start()

        mv, mi = accs[0], (accs[1] * U + 0) * 16 + lanes
        for k in range(1, U):
            v, i = accs[2 * k], (accs[2 * k + 1] * U + k) * 16 + lanes
            take = jnp.logical_or(v > mv, jnp.logical_and(v == mv, i < mi))
            mv = jnp.where(take, v, mv)
            mi = jnp.where(take, i, mi)
        maxv = mv
        for s in (8, 4, 2, 1):
            maxv = jnp.maximum(maxv, _xlane(maxv, lanes ^ s))
        cand = jnp.where(mv == maxv, mi, INT_MAX)
        for s in (8, 4, 2, 1):
            cand = jnp.minimum(cand, _xlane(cand, lanes ^ s))

        pos = j % P
        pos0 = jnp.where(j < P, pos, 16)
        pos1 = jnp.where(j >= P, pos, 16)
        ids0 = jnp.where(lanes == pos0, cand, ids0)
        ids1 = jnp.where(lanes == pos1, cand, ids1)
        return ids0, ids1

    zeros = jnp.zeros((16,), dtype=jnp.int32)
    ids = lax.fori_loop(0, RPW, row_body, (zeros, zeros))

    for b in range(BPW):
        spec_b = spec_v[pl.ds(8 * b, 16)]
        mismatch = jnp.logical_or(spec_b != ids[b], lanes >= S)
        first_mm = jnp.where(mismatch, lanes, 16)
        for s in (8, 4, 2, 1):
            first_mm = jnp.minimum(first_mm, _xlane(first_mm, lanes ^ s))
        tokens = jnp.where(lanes < first_mm + 1, ids[b], -1)
        out_v[pl.ds(OUTW * b, 16)] = tokens

    pltpu.sync_copy(out_v, out_hbm.at[pl.ds(wid * BPW * OUTW, BPW * OUTW)])


@jax.jit
def kernel(logits, spec_token_ids):
    mesh = plsc.VectorSubcoreMesh(core_axis_name="c", subcore_axis_name="s")
    run = functools.partial(
        pl.kernel,
        mesh=mesh,
        out_type=jax.ShapeDtypeStruct((B * OUTW,), jnp.int32),
        scratch_types=[
            pltpu.VMEM((C,), jnp.float32),
            pltpu.VMEM((C,), jnp.float32),
            pltpu.VMEM((24,), jnp.int32),
            pltpu.VMEM((BPW * OUTW,), jnp.int32),
            pltpu.SemaphoreType.DMA,
            pltpu.SemaphoreType.DMA,
        ],
    )(_sc_kernel)
    out = run(logits.reshape(-1), spec_token_ids.astype(jnp.int32).reshape(-1))
    return out.reshape(B, OUTW)[:, :P]


# speedup vs baseline: 1.0062x; 1.0062x over previous
import functools

import jax
import jax.numpy as jnp
from jax import lax
from jax.experimental import pallas as pl
from jax.experimental.pallas import tpu as pltpu
from jax.experimental.pallas import tpu_sc as plsc

B = 64
S = 8
V = 100000
P = S + 1
R = B * P
NW = 32
RSC = 128
RPW = RSC // NW
RTC = R - RSC
TCBH = 64
BK = 12800
NBK = (V + BK - 1) // BK
C = V // 2
U = 5
NSTEP = C // 16 // U
NEG_INF = float("-inf")
INT_MAX = 0x7FFFFFFF


def _xlane(x, idx):
    dn = lax.GatherDimensionNumbers(
        offset_dims=(), collapsed_slice_dims=(0,), start_index_map=(0,))
    return lax.gather(x, idx[:, None], dn, slice_sizes=(1,),
                      mode=lax.GatherScatterMode.PROMISE_IN_BOUNDS)


def _scan_chunk(buf, q, accs):
    @plsc.parallel_loop(0, NSTEP, step=1, unroll=8, carry=accs)
    def body(i, accs):
        gv = jnp.full((16,), q * NSTEP + i, dtype=jnp.int32)
        out = []
        for k in range(U):
            bv, bt = accs[2 * k], accs[2 * k + 1]
            v = buf[pl.ds((i * U + k) * 16, 16)]
            upd = v > bv
            bv = jnp.maximum(v, bv)
            bt = jnp.where(upd, gv, bt)
            out += [bv, bt]
        return tuple(out)

    return body


def _sc_kernel(logits_hbm, out_hbm, buf0, buf1, out_v, sem0, sem1):
    wid = lax.axis_index("s") * 2 + lax.axis_index("c")
    lanes = lax.iota(jnp.int32, 16)
    bufs = (buf0, buf1)
    sems = (sem0, sem1)

    def dma(j, q):
        base = (wid * RPW + j) * V + q * C
        return pltpu.make_async_copy(
            logits_hbm.at[pl.ds(base, C)], bufs[q], sems[q])

    dma(0, 0).start()
    dma(0, 1).start()

    def row_body(j, ids):
        accs = ()
        for _ in range(U):
            accs += (jnp.full((16,), NEG_INF, dtype=jnp.float32),
                     jnp.zeros((16,), dtype=jnp.int32))
        for q in range(2):
            dma(j, q).wait()
            accs = _scan_chunk(bufs[q], q, accs)

            @pl.when(j < RPW - 1)
            def _():
                dma(j + 1, q).start()

        mv, mi = accs[0], (accs[1] * U + 0) * 16 + lanes
        for k in range(1, U):
            v, i = accs[2 * k], (accs[2 * k + 1] * U + k) * 16 + lanes
            take = jnp.logical_or(v > mv, jnp.logical_and(v == mv, i < mi))
            mv = jnp.where(take, v, mv)
            mi = jnp.where(take, i, mi)
        maxv = mv
        for s in (8, 4, 2, 1):
            maxv = jnp.maximum(maxv, _xlane(maxv, lanes ^ s))
        cand = jnp.where(mv == maxv, mi, INT_MAX)
        for s in (8, 4, 2, 1):
            cand = jnp.minimum(cand, _xlane(cand, lanes ^ s))

        return jnp.where(lanes == j, cand, ids)

    ids = lax.fori_loop(0, RPW, row_body, jnp.zeros((16,), dtype=jnp.int32))
    out_v[pl.ds(0, 16)] = ids
    pltpu.sync_copy(out_v, out_hbm.at[pl.ds(wid * 16, 16)])


def _tc_argmax_body(x_ref, out_ref, bv_ref):
    vb = pl.program_id(1)
    col = lax.broadcasted_iota(jnp.int32, x_ref.shape, 1) + vb * BK
    x = jnp.where(col < V, x_ref[...], NEG_INF)
    bmax = jnp.max(x, axis=1, keepdims=True)
    bidx = jnp.min(jnp.where(x == bmax, col, INT_MAX), axis=1, keepdims=True)

    @pl.when(vb == 0)
    def _():
        bv_ref[...] = bmax
        out_ref[...] = bidx

    @pl.when(vb > 0)
    def _():
        upd = bmax > bv_ref[...]
        out_ref[...] = jnp.where(upd, bidx, out_ref[...])
        bv_ref[...] = jnp.where(upd, bmax, bv_ref[...])


def _tc_accept_body(ids_ref, spec_ref, out_ref):
    ids = ids_ref[...]
    spec = spec_ref[...]
    pos8 = lax.broadcasted_iota(jnp.int32, spec.shape, 1)
    pos9 = lax.broadcasted_iota(jnp.int32, ids.shape, 1)
    fm = jnp.min(jnp.where(spec != ids[:, :S], pos8, S), axis=1, keepdims=True)
    out_ref[...] = jnp.where(pos9 <= fm, ids, -1)


@jax.jit
def kernel(logits, spec_token_ids):
    spec = spec_token_ids.astype(jnp.int32)

    mesh = plsc.VectorSubcoreMesh(core_axis_name="c", subcore_axis_name="s")
    sc_run = functools.partial(
        pl.kernel,
        mesh=mesh,
        out_type=jax.ShapeDtypeStruct((NW * 16,), jnp.int32),
        scratch_types=[
            pltpu.VMEM((C,), jnp.float32),
            pltpu.VMEM((C,), jnp.float32),
            pltpu.VMEM((16,), jnp.int32),
            pltpu.SemaphoreType.DMA,
            pltpu.SemaphoreType.DMA,
        ],
    )(_sc_kernel)
    sc_ids = sc_run(logits.reshape(-1))
    sc_ids = sc_ids.reshape(NW, 16)[:, :RPW].reshape(RSC)

    tc_ids = pl.pallas_call(
        _tc_argmax_body,
        grid=(RTC // TCBH, NBK),
        in_specs=[pl.BlockSpec((TCBH, BK),
                               lambda i, j: (RSC // TCBH + i, j))],
        out_specs=pl.BlockSpec((TCBH, 1), lambda i, j: (i, 0)),
        out_shape=jax.ShapeDtypeStruct((RTC, 1), jnp.int32),
        scratch_shapes=[pltpu.VMEM((TCBH, 1), jnp.float32)],
        compiler_params=pltpu.CompilerParams(
            dimension_semantics=("parallel", "arbitrary")),
    )(logits)

    ids = jnp.concatenate([sc_ids, tc_ids.reshape(RTC)]).reshape(B, P)

    return pl.pallas_call(
        _tc_accept_body,
        in_specs=[pl.BlockSpec((B, P), lambda: (0, 0)),
                  pl.BlockSpec((B, S), lambda: (0, 0))],
        out_specs=pl.BlockSpec((B, P), lambda: (0, 0)),
        out_shape=jax.ShapeDtypeStruct((B, P), jnp.int32),
    )(ids, spec)


# speedup vs baseline: 1.0173x; 1.0110x over previous
import functools

import jax
import jax.numpy as jnp
from jax import lax
from jax.experimental import pallas as pl
from jax.experimental.pallas import tpu as pltpu
from jax.experimental.pallas import tpu_sc as plsc

B = 64
S = 8
V = 100000
P = S + 1
R = B * P
NW = 32
RSC = 128
RPW = RSC // NW
RTC = R - RSC
TCBH = 64
BK = 12800
NBK = (V + BK - 1) // BK
NL = 128
SPB = BK // NL
VALID_LAST = V - (NBK - 1) * BK
FULL_LAST = VALID_LAST // NL
TAIL_LANES = VALID_LAST - FULL_LAST * NL
C = V // 2
U = 5
NSTEP = C // 16 // U
NEG_INF = float("-inf")
INT_MAX = 0x7FFFFFFF


def _xlane(x, idx):
    dn = lax.GatherDimensionNumbers(
        offset_dims=(), collapsed_slice_dims=(0,), start_index_map=(0,))
    return lax.gather(x, idx[:, None], dn, slice_sizes=(1,),
                      mode=lax.GatherScatterMode.PROMISE_IN_BOUNDS)


def _scan_chunk(buf, q, accs):
    @plsc.parallel_loop(0, NSTEP, step=1, unroll=8, carry=accs)
    def body(i, accs):
        gv = jnp.full((16,), q * NSTEP + i, dtype=jnp.int32)
        out = []
        for k in range(U):
            bv, bt = accs[2 * k], accs[2 * k + 1]
            v = buf[pl.ds((i * U + k) * 16, 16)]
            upd = v > bv
            bv = jnp.maximum(v, bv)
            bt = jnp.where(upd, gv, bt)
            out += [bv, bt]
        return tuple(out)

    return body


def _sc_kernel(logits_hbm, out_hbm, buf0, buf1, out_v, sem0, sem1):
    wid = lax.axis_index("s") * 2 + lax.axis_index("c")
    lanes = lax.iota(jnp.int32, 16)
    bufs = (buf0, buf1)
    sems = (sem0, sem1)

    def dma(j, q):
        base = (wid * RPW + j) * V + q * C
        return pltpu.make_async_copy(
            logits_hbm.at[pl.ds(base, C)], bufs[q], sems[q])

    dma(0, 0).start()
    dma(0, 1).start()

    def row_body(j, ids):
        accs = ()
        for _ in range(U):
            accs += (jnp.full((16,), NEG_INF, dtype=jnp.float32),
                     jnp.zeros((16,), dtype=jnp.int32))
        for q in range(2):
            dma(j, q).wait()
            accs = _scan_chunk(bufs[q], q, accs)

            @pl.when(j < RPW - 1)
            def _():
                dma(j + 1, q).start()

        mv, mi = accs[0], (accs[1] * U + 0) * 16 + lanes
        for k in range(1, U):
            v, i = accs[2 * k], (accs[2 * k + 1] * U + k) * 16 + lanes
            take = jnp.logical_or(v > mv, jnp.logical_and(v == mv, i < mi))
            mv = jnp.where(take, v, mv)
            mi = jnp.where(take, i, mi)
        maxv = mv
        for s in (8, 4, 2, 1):
            maxv = jnp.maximum(maxv, _xlane(maxv, lanes ^ s))
        cand = jnp.where(mv == maxv, mi, INT_MAX)
        for s in (8, 4, 2, 1):
            cand = jnp.minimum(cand, _xlane(cand, lanes ^ s))

        return jnp.where(lanes == j, cand, ids)

    ids = lax.fori_loop(0, RPW, row_body, jnp.zeros((16,), dtype=jnp.int32))
    out_v[pl.ds(0, 16)] = ids
    pltpu.sync_copy(out_v, out_hbm.at[pl.ds(wid * 16, 16)])


def _tc_argmax_body(x_ref, out_ref, bv_ref, bi_ref):
    vb = pl.program_id(1)

    @pl.when(vb == 0)
    def _():
        bv_ref[...] = jnp.full(bv_ref.shape, NEG_INF, dtype=jnp.float32)
        bi_ref[...] = jnp.zeros(bi_ref.shape, dtype=jnp.int32)

    def scan(nsteps, tail):
        bv = bv_ref[...]
        bi = bi_ref[...]
        lanes = lax.broadcasted_iota(jnp.int32, bv.shape, 1)
        for k in range(nsteps + tail):
            v = x_ref[:, k * NL:(k + 1) * NL]
            if k >= nsteps:
                v = jnp.where(lanes < TAIL_LANES, v, NEG_INF)
            upd = v > bv
            bv = jnp.where(upd, v, bv)
            bi = jnp.where(upd, vb * SPB + k, bi)
        bv_ref[...] = bv
        bi_ref[...] = bi

    @pl.when(vb < NBK - 1)
    def _():
        scan(SPB, 0)

    @pl.when(vb == NBK - 1)
    def _():
        scan(FULL_LAST, 1)
        bv = bv_ref[...]
        lanes = lax.broadcasted_iota(jnp.int32, bv.shape, 1)
        rmax = jnp.max(bv, axis=1, keepdims=True)
        idx = bi_ref[...] * NL + lanes
        cand = jnp.where(bv == rmax, idx, INT_MAX)
        out_ref[...] = jnp.min(cand, axis=1, keepdims=True)


def _tc_accept_body(ids_ref, spec_ref, out_ref):
    ids = ids_ref[...]
    spec = spec_ref[...]
    pos8 = lax.broadcasted_iota(jnp.int32, spec.shape, 1)
    pos9 = lax.broadcasted_iota(jnp.int32, ids.shape, 1)
    fm = jnp.min(jnp.where(spec != ids[:, :S], pos8, S), axis=1, keepdims=True)
    out_ref[...] = jnp.where(pos9 <= fm, ids, -1)


@jax.jit
def kernel(logits, spec_token_ids):
    spec = spec_token_ids.astype(jnp.int32)

    mesh = plsc.VectorSubcoreMesh(core_axis_name="c", subcore_axis_name="s")
    sc_run = functools.partial(
        pl.kernel,
        mesh=mesh,
        out_type=jax.ShapeDtypeStruct((NW * 16,), jnp.int32),
        scratch_types=[
            pltpu.VMEM((C,), jnp.float32),
            pltpu.VMEM((C,), jnp.float32),
            pltpu.VMEM((16,), jnp.int32),
            pltpu.SemaphoreType.DMA,
            pltpu.SemaphoreType.DMA,
        ],
    )(_sc_kernel)
    sc_ids = sc_run(logits.reshape(-1))
    sc_ids = sc_ids.reshape(NW, 16)[:, :RPW].reshape(RSC)

    tc_ids = pl.pallas_call(
        _tc_argmax_body,
        grid=(RTC // TCBH, NBK),
        in_specs=[pl.BlockSpec((TCBH, BK),
                               lambda i, j: (RSC // TCBH + i, j))],
        out_specs=pl.BlockSpec((TCBH, 1), lambda i, j: (i, 0)),
        out_shape=jax.ShapeDtypeStruct((RTC, 1), jnp.int32),
        scratch_shapes=[pltpu.VMEM((TCBH, NL), jnp.float32),
                        pltpu.VMEM((TCBH, NL), jnp.int32)],
        compiler_params=pltpu.CompilerParams(
            dimension_semantics=("parallel", "arbitrary")),
    )(logits)

    ids = jnp.concatenate([sc_ids, tc_ids.reshape(RTC)]).reshape(B, P)

    return pl.pallas_call(
        _tc_accept_body,
        in_specs=[pl.BlockSpec((B, P), lambda: (0, 0)),
                  pl.BlockSpec((B, S), lambda: (0, 0))],
        out_specs=pl.BlockSpec((B, P), lambda: (0, 0)),
        out_shape=jax.ShapeDtypeStruct((B, P), jnp.int32),
    )(ids, spec)


# speedup vs baseline: 4.0529x; 3.9839x over previous
import functools

import jax
import jax.numpy as jnp
from jax import lax
from jax.experimental import pallas as pl
from jax.experimental.pallas import tpu as pltpu
from jax.experimental.pallas import tpu_sc as plsc

B = 64
S = 8
V = 100000
P = S + 1
R = B * P
NW = 32
RSC = 0
RPW = max(RSC // NW, 1)
RTC = R - RSC
TCBH = 64
BK = 12800
NBK = (V + BK - 1) // BK
NL = 128
SPB = BK // NL
VALID_LAST = V - (NBK - 1) * BK
FULL_LAST = VALID_LAST // NL
TAIL_LANES = VALID_LAST - FULL_LAST * NL
C = V // 2
U = 5
NSTEP = C // 16 // U
NEG_INF = float("-inf")
INT_MAX = 0x7FFFFFFF


def _xlane(x, idx):
    dn = lax.GatherDimensionNumbers(
        offset_dims=(), collapsed_slice_dims=(0,), start_index_map=(0,))
    return lax.gather(x, idx[:, None], dn, slice_sizes=(1,),
                      mode=lax.GatherScatterMode.PROMISE_IN_BOUNDS)


def _scan_chunk(buf, q, accs):
    @plsc.parallel_loop(0, NSTEP, step=1, unroll=8, carry=accs)
    def body(i, accs):
        gv = jnp.full((16,), q * NSTEP + i, dtype=jnp.int32)
        out = []
        for k in range(U):
            bv, bt = accs[2 * k], accs[2 * k + 1]
            v = buf[pl.ds((i * U + k) * 16, 16)]
            upd = v > bv
            bv = jnp.maximum(v, bv)
            bt = jnp.where(upd, gv, bt)
            out += [bv, bt]
        return tuple(out)

    return body


def _sc_kernel(logits_hbm, out_hbm, buf0, buf1, out_v, sem0, sem1):
    wid = lax.axis_index("s") * 2 + lax.axis_index("c")
    lanes = lax.iota(jnp.int32, 16)
    bufs = (buf0, buf1)
    sems = (sem0, sem1)

    def dma(j, q):
        base = (wid * RPW + j) * V + q * C
        return pltpu.make_async_copy(
            logits_hbm.at[pl.ds(base, C)], bufs[q], sems[q])

    dma(0, 0).start()
    dma(0, 1).start()

    def row_body(j, ids):
        accs = ()
        for _ in range(U):
            accs += (jnp.full((16,), NEG_INF, dtype=jnp.float32),
                     jnp.zeros((16,), dtype=jnp.int32))
        for q in range(2):
            dma(j, q).wait()
            accs = _scan_chunk(bufs[q], q, accs)

            @pl.when(j < RPW - 1)
            def _():
                dma(j + 1, q).start()

        mv, mi = accs[0], (accs[1] * U + 0) * 16 + lanes
        for k in range(1, U):
            v, i = accs[2 * k], (accs[2 * k + 1] * U + k) * 16 + lanes
            take = jnp.logical_or(v > mv, jnp.logical_and(v == mv, i < mi))
            mv = jnp.where(take, v, mv)
            mi = jnp.where(take, i, mi)
        maxv = mv
        for s in (8, 4, 2, 1):
            maxv = jnp.maximum(maxv, _xlane(maxv, lanes ^ s))
        cand = jnp.where(mv == maxv, mi, INT_MAX)
        for s in (8, 4, 2, 1):
            cand = jnp.minimum(cand, _xlane(cand, lanes ^ s))

        return jnp.where(lanes == j, cand, ids)

    ids = lax.fori_loop(0, RPW, row_body, jnp.zeros((16,), dtype=jnp.int32))
    out_v[pl.ds(0, 16)] = ids
    pltpu.sync_copy(out_v, out_hbm.at[pl.ds(wid * 16, 16)])


def _tc_argmax_body(x_ref, out_ref, bv_ref, bi_ref):
    vb = pl.program_id(1)

    @pl.when(vb == 0)
    def _():
        bv_ref[...] = jnp.full(bv_ref.shape, NEG_INF, dtype=jnp.float32)
        bi_ref[...] = jnp.zeros(bi_ref.shape, dtype=jnp.int32)

    def scan(nsteps, tail):
        bv = bv_ref[...]
        bi = bi_ref[...]
        lanes = lax.broadcasted_iota(jnp.int32, bv.shape, 1)
        for k in range(nsteps + tail):
            v = x_ref[:, k * NL:(k + 1) * NL]
            if k >= nsteps:
                v = jnp.where(lanes < TAIL_LANES, v, NEG_INF)
            upd = v > bv
            bv = jnp.where(upd, v, bv)
            bi = jnp.where(upd, vb * SPB + k, bi)
        bv_ref[...] = bv
        bi_ref[...] = bi

    @pl.when(vb < NBK - 1)
    def _():
        scan(SPB, 0)

    @pl.when(vb == NBK - 1)
    def _():
        scan(FULL_LAST, 1)
        bv = bv_ref[...]
        lanes = lax.broadcasted_iota(jnp.int32, bv.shape, 1)
        rmax = jnp.max(bv, axis=1, keepdims=True)
        idx = bi_ref[...] * NL + lanes
        cand = jnp.where(bv == rmax, idx, INT_MAX)
        out_ref[...] = jnp.min(cand, axis=1, keepdims=True)


def _tc_accept_body(ids_ref, spec_ref, out_ref):
    ids = ids_ref[...]
    spec = spec_ref[...]
    pos8 = lax.broadcasted_iota(jnp.int32, spec.shape, 1)
    pos9 = lax.broadcasted_iota(jnp.int32, ids.shape, 1)
    fm = jnp.min(jnp.where(spec != ids[:, :S], pos8, S), axis=1, keepdims=True)
    out_ref[...] = jnp.where(pos9 <= fm, ids, -1)


@jax.jit
def kernel(logits, spec_token_ids):
    spec = spec_token_ids.astype(jnp.int32)

    if RSC:
        mesh = plsc.VectorSubcoreMesh(core_axis_name="c", subcore_axis_name="s")
        sc_run = functools.partial(
            pl.kernel,
            mesh=mesh,
            out_type=jax.ShapeDtypeStruct((NW * 16,), jnp.int32),
            scratch_types=[
                pltpu.VMEM((C,), jnp.float32),
                pltpu.VMEM((C,), jnp.float32),
                pltpu.VMEM((16,), jnp.int32),
                pltpu.SemaphoreType.DMA,
                pltpu.SemaphoreType.DMA,
            ],
        )(_sc_kernel)
        sc_ids = sc_run(logits.reshape(-1))
        sc_ids = [sc_ids.reshape(NW, 16)[:, :RPW].reshape(RSC)]
    else:
        sc_ids = []

    tc_ids = pl.pallas_call(
        _tc_argmax_body,
        grid=(RTC // TCBH, NBK),
        in_specs=[pl.BlockSpec((TCBH, BK),
                               lambda i, j: (RSC // TCBH + i, j))],
        out_specs=pl.BlockSpec((TCBH, 1), lambda i, j: (i, 0)),
        out_shape=jax.ShapeDtypeStruct((RTC, 1), jnp.int32),
        scratch_shapes=[pltpu.VMEM((TCBH, NL), jnp.float32),
                        pltpu.VMEM((TCBH, NL), jnp.int32)],
        compiler_params=pltpu.CompilerParams(
            dimension_semantics=("parallel", "arbitrary")),
    )(logits)

    ids = jnp.concatenate(sc_ids + [tc_ids.reshape(RTC)]).reshape(B, P)

    return pl.pallas_call(
        _tc_accept_body,
        in_specs=[pl.BlockSpec((B, P), lambda: (0, 0)),
                  pl.BlockSpec((B, S), lambda: (0, 0))],
        out_specs=pl.BlockSpec((B, P), lambda: (0, 0)),
        out_shape=jax.ShapeDtypeStruct((B, P), jnp.int32),
    )(ids, spec)


# speedup vs baseline: 4.1979x; 1.0358x over previous
import functools

import jax
import jax.numpy as jnp
from jax import lax
from jax.experimental import pallas as pl
from jax.experimental.pallas import tpu as pltpu
from jax.experimental.pallas import tpu_sc as plsc

B = 64
S = 8
V = 100000
P = S + 1
R = B * P
NW = 32
GR = 8
RSC = NW * GR
RTC = R - RSC
TCBH = 64
BK = 12800
NBK = (V + BK - 1) // BK
NL = 128
SPB = BK // NL
VALID_LAST = V - (NBK - 1) * BK
FULL_LAST = VALID_LAST // NL
TAIL_LANES = VALID_LAST - FULL_LAST * NL
CW = 6400
VSC = (V // NL) * NL
NCH = 16
CWS = [CW] * (NCH - 1) + [VSC - (NCH - 1) * CW]
CBASE = [c * CW // 16 for c in range(NCH)]
NEG_INF = float("-inf")
INT_MAX = 0x7FFFFFFF


def _xlane(x, idx):
    dn = lax.GatherDimensionNumbers(
        offset_dims=(), collapsed_slice_dims=(0,), start_index_map=(0,))
    return lax.gather(x, idx[:, None], dn, slice_sizes=(1,),
                      mode=lax.GatherScatterMode.PROMISE_IN_BOUNDS)


def _scan_chunk(buf, c, accs):
    @plsc.parallel_loop(0, CWS[c] // 16, step=1, unroll=4, carry=accs)
    def body(i, accs):
        gv = jnp.full((16,), CBASE[c] + i, dtype=jnp.int32)
        out = []
        for ri in range(GR):
            bv, bt = accs[2 * ri], accs[2 * ri + 1]
            v = buf[ri, pl.ds(i * 16, 16)]
            upd = v > bv
            bv = jnp.maximum(v, bv)
            bt = jnp.where(upd, gv, bt)
            out += [bv, bt]
        return tuple(out)

    return body


def _sc_kernel(logits_hbm, outv_hbm, outi_hbm, buf0, buf1, out_v, out_i,
               sem0, sem1):
    wid = lax.axis_index("s") * 2 + lax.axis_index("c")
    lanes = lax.iota(jnp.int32, 16)
    bufs = (buf0, buf1)
    sems = (sem0, sem1)

    def dma(c):
        w = CWS[c]
        return pltpu.make_async_copy(
            logits_hbm.at[pl.ds(wid * GR, GR), pl.ds(c * CW, w)],
            bufs[c % 2].at[:, pl.ds(0, w)], sems[c % 2])

    accs = ()
    for _ in range(GR):
        accs += (jnp.full((16,), NEG_INF, dtype=jnp.float32),
                 jnp.zeros((16,), dtype=jnp.int32))

    dma(0).start()
    dma(1).start()
    for c in range(NCH):
        dma(c).wait()
        accs = _scan_chunk(bufs[c % 2], c, accs)
        if c + 2 < NCH:
            dma(c + 2).start()

    res_v = jnp.zeros((16,), dtype=jnp.float32)
    res_i = jnp.zeros((16,), dtype=jnp.int32)
    for ri in range(GR):
        mv, mi = accs[2 * ri], accs[2 * ri + 1] * 16 + lanes
        maxv = mv
        for s in (8, 4, 2, 1):
            maxv = jnp.maximum(maxv, _xlane(maxv, lanes ^ s))
        cand = jnp.where(mv == maxv, mi, INT_MAX)
        for s in (8, 4, 2, 1):
            cand = jnp.minimum(cand, _xlane(cand, lanes ^ s))
        res_v = jnp.where(lanes == ri, maxv, res_v)
        res_i = jnp.where(lanes == ri, cand, res_i)

    out_v[pl.ds(0, 16)] = res_v
    out_i[pl.ds(0, 16)] = res_i
    pltpu.sync_copy(out_v, outv_hbm.at[pl.ds(wid * 16, 16)])
    pltpu.sync_copy(out_i, outi_hbm.at[pl.ds(wid * 16, 16)])


def _tc_argmax_body(x_ref, out_ref, bv_ref, bi_ref):
    vb = pl.program_id(1)

    @pl.when(vb == 0)
    def _():
        bv_ref[...] = jnp.full(bv_ref.shape, NEG_INF, dtype=jnp.float32)
        bi_ref[...] = jnp.zeros(bi_ref.shape, dtype=jnp.int32)

    def scan(nsteps, tail):
        bv = bv_ref[...]
        bi = bi_ref[...]
        lanes = lax.broadcasted_iota(jnp.int32, bv.shape, 1)
        for k in range(nsteps + tail):
            v = x_ref[:, k * NL:(k + 1) * NL]
            if k >= nsteps:
                v = jnp.where(lanes < TAIL_LANES, v, NEG_INF)
            upd = v > bv
            bv = jnp.where(upd, v, bv)
            bi = jnp.where(upd, vb * SPB + k, bi)
        bv_ref[...] = bv
        bi_ref[...] = bi

    @pl.when(vb < NBK - 1)
    def _():
        scan(SPB, 0)

    @pl.when(vb == NBK - 1)
    def _():
        scan(FULL_LAST, 1)
        bv = bv_ref[...]
        lanes = lax.broadcasted_iota(jnp.int32, bv.shape, 1)
        rmax = jnp.max(bv, axis=1, keepdims=True)
        idx = bi_ref[...] * NL + lanes
        cand = jnp.where(bv == rmax, idx, INT_MAX)
        out_ref[...] = jnp.min(cand, axis=1, keepdims=True)


def _tc_tail_body(x_ref, scv_ref, sci_ref, out_ref):
    lanes = lax.broadcasted_iota(jnp.int32, x_ref.shape, 1)
    x = jnp.where(lanes < TAIL_LANES, x_ref[...], NEG_INF)
    rmax = jnp.max(x, axis=1, keepdims=True)
    ridx = jnp.min(jnp.where(x == rmax, VSC + lanes, INT_MAX),
                   axis=1, keepdims=True)
    upd = rmax > scv_ref[...]
    out_ref[...] = jnp.where(upd, ridx, sci_ref[...])


def _tc_accept_body(ids_ref, spec_ref, out_ref):
    ids = ids_ref[...]
    spec = spec_ref[...]
    pos8 = lax.broadcasted_iota(jnp.int32, spec.shape, 1)
    pos9 = lax.broadcasted_iota(jnp.int32, ids.shape, 1)
    fm = jnp.min(jnp.where(spec != ids[:, :S], pos8, S), axis=1, keepdims=True)
    out_ref[...] = jnp.where(pos9 <= fm, ids, -1)


@jax.jit
def kernel(logits, spec_token_ids):
    spec = spec_token_ids.astype(jnp.int32)

    mesh = plsc.VectorSubcoreMesh(core_axis_name="c", subcore_axis_name="s")
    sc_run = functools.partial(
        pl.kernel,
        mesh=mesh,
        out_type=(jax.ShapeDtypeStruct((NW * 16,), jnp.float32),
                  jax.ShapeDtypeStruct((NW * 16,), jnp.int32)),
        scratch_types=[
            pltpu.VMEM((GR, CW), jnp.float32),
            pltpu.VMEM((GR, CW), jnp.float32),
            pltpu.VMEM((16,), jnp.float32),
            pltpu.VMEM((16,), jnp.int32),
            pltpu.SemaphoreType.DMA,
            pltpu.SemaphoreType.DMA,
        ],
    )(_sc_kernel)
    sc_v, sc_i = sc_run(logits)
    sc_v = sc_v.reshape(NW, 16)[:, :GR].reshape(RSC, 1)
    sc_i = sc_i.reshape(NW, 16)[:, :GR].reshape(RSC, 1)

    sc_ids = pl.pallas_call(
        _tc_tail_body,
        grid=(1,),
        in_specs=[pl.BlockSpec((RSC, NL), lambda i: (0, VSC // NL)),
                  pl.BlockSpec((RSC, 1), lambda i: (0, 0)),
                  pl.BlockSpec((RSC, 1), lambda i: (0, 0))],
        out_specs=pl.BlockSpec((RSC, 1), lambda i: (0, 0)),
        out_shape=jax.ShapeDtypeStruct((RSC, 1), jnp.int32),
    )(logits, sc_v, sc_i)
    sc_ids = sc_ids.reshape(RSC)

    tc_ids = pl.pallas_call(
        _tc_argmax_body,
        grid=(RTC // TCBH, NBK),
        in_specs=[pl.BlockSpec((TCBH, BK),
                               lambda i, j: (RSC // TCBH + i, j))],
        out_specs=pl.BlockSpec((TCBH, 1), lambda i, j: (i, 0)),
        out_shape=jax.ShapeDtypeStruct((RTC, 1), jnp.int32),
        scratch_shapes=[pltpu.VMEM((TCBH, NL), jnp.float32),
                        pltpu.VMEM((TCBH, NL), jnp.int32)],
        compiler_params=pltpu.CompilerParams(
            dimension_semantics=("parallel", "arbitrary")),
    )(logits)

    ids = jnp.concatenate([sc_ids, tc_ids.reshape(RTC)]).reshape(B, P)

    return pl.pallas_call(
        _tc_accept_body,
        in_specs=[pl.BlockSpec((B, P), lambda: (0, 0)),
                  pl.BlockSpec((B, S), lambda: (0, 0))],
        out_specs=pl.BlockSpec((B, P), lambda: (0, 0)),
        out_shape=jax.ShapeDtypeStruct((B, P), jnp.int32),
    )(ids, spec)


# speedup vs baseline: 4.2979x; 1.0238x over previous
import functools

import jax
import jax.numpy as jnp
from jax import lax
from jax.experimental import pallas as pl
from jax.experimental.pallas import tpu as pltpu
from jax.experimental.pallas import tpu_sc as plsc

B = 64
S = 8
V = 100000
P = S + 1
R = B * P
NW = 32
GR = 8
RSC = NW * GR
RTC = R - RSC
TCBH = 64
BK = 25600
NBK = (V + BK - 1) // BK
NL = 128
SPB = BK // NL
VALID_LAST = V - (NBK - 1) * BK
FULL_LAST = VALID_LAST // NL
TAIL_LANES = VALID_LAST - FULL_LAST * NL
CW = 6400
VSC = (V // NL) * NL
NCH = 16
CWS = [CW] * (NCH - 1) + [VSC - (NCH - 1) * CW]
CBASE = [c * CW // 16 for c in range(NCH)]
NEG_INF = float("-inf")
INT_MAX = 0x7FFFFFFF


def _xlane(x, idx):
    dn = lax.GatherDimensionNumbers(
        offset_dims=(), collapsed_slice_dims=(0,), start_index_map=(0,))
    return lax.gather(x, idx[:, None], dn, slice_sizes=(1,),
                      mode=lax.GatherScatterMode.PROMISE_IN_BOUNDS)


def _scan_chunk(buf, c, accs):
    @plsc.parallel_loop(0, CWS[c] // 16, step=1, unroll=4, carry=accs)
    def body(i, accs):
        gv = jnp.full((16,), CBASE[c] + i, dtype=jnp.int32)
        out = []
        for ri in range(GR):
            bv, bt = accs[2 * ri], accs[2 * ri + 1]
            v = buf[ri, pl.ds(i * 16, 16)]
            upd = v > bv
            bv = jnp.maximum(v, bv)
            bt = jnp.where(upd, gv, bt)
            out += [bv, bt]
        return tuple(out)

    return body


def _sc_kernel(logits_hbm, outv_hbm, outi_hbm, buf0, buf1, out_v, out_i,
               sem0, sem1):
    wid = lax.axis_index("s") * 2 + lax.axis_index("c")
    lanes = lax.iota(jnp.int32, 16)
    bufs = (buf0, buf1)
    sems = (sem0, sem1)

    def dma(c):
        w = CWS[c]
        return pltpu.make_async_copy(
            logits_hbm.at[pl.ds(wid * GR, GR), pl.ds(c * CW, w)],
            bufs[c % 2].at[:, pl.ds(0, w)], sems[c % 2])

    accs = ()
    for _ in range(GR):
        accs += (jnp.full((16,), NEG_INF, dtype=jnp.float32),
                 jnp.zeros((16,), dtype=jnp.int32))

    dma(0).start()
    dma(1).start()
    for c in range(NCH):
        dma(c).wait()
        accs = _scan_chunk(bufs[c % 2], c, accs)
        if c + 2 < NCH:
            dma(c + 2).start()

    res_v = jnp.zeros((16,), dtype=jnp.float32)
    res_i = jnp.zeros((16,), dtype=jnp.int32)
    for ri in range(GR):
        mv, mi = accs[2 * ri], accs[2 * ri + 1] * 16 + lanes
        maxv = mv
        for s in (8, 4, 2, 1):
            maxv = jnp.maximum(maxv, _xlane(maxv, lanes ^ s))
        cand = jnp.where(mv == maxv, mi, INT_MAX)
        for s in (8, 4, 2, 1):
            cand = jnp.minimum(cand, _xlane(cand, lanes ^ s))
        res_v = jnp.where(lanes == ri, maxv, res_v)
        res_i = jnp.where(lanes == ri, cand, res_i)

    out_v[pl.ds(0, 16)] = res_v
    out_i[pl.ds(0, 16)] = res_i
    pltpu.sync_copy(out_v, outv_hbm.at[pl.ds(wid * 16, 16)])
    pltpu.sync_copy(out_i, outi_hbm.at[pl.ds(wid * 16, 16)])


def _tc_argmax_body(x_ref, out_ref, bv_ref, bi_ref):
    vb = pl.program_id(1)

    @pl.when(vb == 0)
    def _():
        bv_ref[...] = jnp.full(bv_ref.shape, NEG_INF, dtype=jnp.float32)
        bi_ref[...] = jnp.zeros(bi_ref.shape, dtype=jnp.int32)

    def scan(nsteps, tail):
        bv = bv_ref[...]
        bi = bi_ref[...]
        lanes = lax.broadcasted_iota(jnp.int32, bv.shape, 1)
        for k in range(nsteps + tail):
            v = x_ref[:, k * NL:(k + 1) * NL]
            if k >= nsteps:
                v = jnp.where(lanes < TAIL_LANES, v, NEG_INF)
            upd = v > bv
            bv = jnp.where(upd, v, bv)
            bi = jnp.where(upd, vb * SPB + k, bi)
        bv_ref[...] = bv
        bi_ref[...] = bi

    @pl.when(vb < NBK - 1)
    def _():
        scan(SPB, 0)

    @pl.when(vb == NBK - 1)
    def _():
        scan(FULL_LAST, 1)
        bv = bv_ref[...]
        lanes = lax.broadcasted_iota(jnp.int32, bv.shape, 1)
        rmax = jnp.max(bv, axis=1, keepdims=True)
        idx = bi_ref[...] * NL + lanes
        cand = jnp.where(bv == rmax, idx, INT_MAX)
        out_ref[...] = jnp.min(cand, axis=1, keepdims=True)


def _tc_tail_body(x_ref, scv_ref, sci_ref, out_ref):
    lanes = lax.broadcasted_iota(jnp.int32, x_ref.shape, 1)
    x = jnp.where(lanes < TAIL_LANES, x_ref[...], NEG_INF)
    rmax = jnp.max(x, axis=1, keepdims=True)
    ridx = jnp.min(jnp.where(x == rmax, VSC + lanes, INT_MAX),
                   axis=1, keepdims=True)
    upd = rmax > scv_ref[...]
    out_ref[...] = jnp.where(upd, ridx, sci_ref[...])


def _tc_accept_body(ids_ref, spec_ref, out_ref):
    ids = ids_ref[...]
    spec = spec_ref[...]
    pos8 = lax.broadcasted_iota(jnp.int32, spec.shape, 1)
    pos9 = lax.broadcasted_iota(jnp.int32, ids.shape, 1)
    fm = jnp.min(jnp.where(spec != ids[:, :S], pos8, S), axis=1, keepdims=True)
    out_ref[...] = jnp.where(pos9 <= fm, ids, -1)


@jax.jit
def kernel(logits, spec_token_ids):
    spec = spec_token_ids.astype(jnp.int32)

    mesh = plsc.VectorSubcoreMesh(core_axis_name="c", subcore_axis_name="s")
    sc_run = functools.partial(
        pl.kernel,
        mesh=mesh,
        out_type=(jax.ShapeDtypeStruct((NW * 16,), jnp.float32),
                  jax.ShapeDtypeStruct((NW * 16,), jnp.int32)),
        scratch_types=[
            pltpu.VMEM((GR, CW), jnp.float32),
            pltpu.VMEM((GR, CW), jnp.float32),
            pltpu.VMEM((16,), jnp.float32),
            pltpu.VMEM((16,), jnp.int32),
            pltpu.SemaphoreType.DMA,
            pltpu.SemaphoreType.DMA,
        ],
    )(_sc_kernel)
    sc_v, sc_i = sc_run(logits)
    sc_v = sc_v.reshape(NW, 16)[:, :GR].reshape(RSC, 1)
    sc_i = sc_i.reshape(NW, 16)[:, :GR].reshape(RSC, 1)

    sc_ids = pl.pallas_call(
        _tc_tail_body,
        grid=(1,),
        in_specs=[pl.BlockSpec((RSC, NL), lambda i: (0, VSC // NL)),
                  pl.BlockSpec((RSC, 1), lambda i: (0, 0)),
                  pl.BlockSpec((RSC, 1), lambda i: (0, 0))],
        out_specs=pl.BlockSpec((RSC, 1), lambda i: (0, 0)),
        out_shape=jax.ShapeDtypeStruct((RSC, 1), jnp.int32),
    )(logits, sc_v, sc_i)
    sc_ids = sc_ids.reshape(RSC)

    tc_ids = pl.pallas_call(
        _tc_argmax_body,
        grid=(RTC // TCBH, NBK),
        in_specs=[pl.BlockSpec((TCBH, BK),
                               lambda i, j: (RSC // TCBH + i, j))],
        out_specs=pl.BlockSpec((TCBH, 1), lambda i, j: (i, 0)),
        out_shape=jax.ShapeDtypeStruct((RTC, 1), jnp.int32),
        scratch_shapes=[pltpu.VMEM((TCBH, NL), jnp.float32),
                        pltpu.VMEM((TCBH, NL), jnp.int32)],
        compiler_params=pltpu.CompilerParams(
            dimension_semantics=("parallel", "arbitrary")),
    )(logits)

    ids = jnp.concatenate([sc_ids, tc_ids.reshape(RTC)]).reshape(B, P)

    return pl.pallas_call(
        _tc_accept_body,
        in_specs=[pl.BlockSpec((B, P), lambda: (0, 0)),
                  pl.BlockSpec((B, S), lambda: (0, 0))],
        out_specs=pl.BlockSpec((B, P), lambda: (0, 0)),
        out_shape=jax.ShapeDtypeStruct((B, P), jnp.int32),
    )(ids, spec)
